# baseline (speedup 1.0000x reference)
import jax
import jax.numpy as jnp
from jax.experimental import pallas as pl
from jax.experimental.pallas import tpu as pltpu

_N = 8192
_D = 64
_CUTOFF = 0.5
_MU = 1.0
_SIGMA = 0.5
_C2 = -1.4426950408889634 / (2.0 * _SIGMA * _SIGMA)

_BR = 256
_NBUF = 2
_NBLK = _N // _BR


def _interact_kernel(z_ref, w_ref, b_ref, dist_hbm, out_ref, buf, msg, sems):
    def copy_in(blk, slot):
        return pltpu.make_async_copy(
            dist_hbm.at[pl.ds(blk * _BR, _BR), :],
            buf.at[slot],
            sems.at[slot],
        )

    for s in range(_NBUF):
        copy_in(s, s).start()

    msg[...] = (
        jnp.dot(z_ref[...], w_ref[...], preferred_element_type=jnp.float32)
        + b_ref[...]
    ).astype(jnp.bfloat16)

    def body(blk, carry):
        slot = jax.lax.rem(blk, _NBUF)
        copy_in(blk, slot).wait()
        r = buf[slot]
        u = 1.0 / r
        t = u - 1.0 / _MU
        w = jnp.where(r < _CUTOFF, jnp.exp2(_C2 * (t * t)), 0.0).astype(jnp.bfloat16)
        out_ref[pl.ds(blk * _BR, _BR), :] = jnp.dot(
            w, msg[...], preferred_element_type=jnp.float32
        )

        @pl.when(blk + _NBUF < _NBLK)
        def _prefetch():
            copy_in(blk + _NBUF, slot).start()

        return carry

    jax.lax.fori_loop(0, _NBLK, body, 0)


def kernel(z, dist_matrix, W, B):
    out = pl.pallas_call(
        _interact_kernel,
        in_specs=[
            pl.BlockSpec((_N, _D), lambda: (0, 0)),
            pl.BlockSpec((_D, _D), lambda: (0, 0)),
            pl.BlockSpec((1, _D), lambda: (0, 0)),
            pl.BlockSpec(memory_space=pltpu.HBM),
        ],
        out_specs=pl.BlockSpec((_N, _D), lambda: (0, 0)),
        out_shape=jax.ShapeDtypeStruct((_N, _D), jnp.float32),
        scratch_shapes=[
            pltpu.VMEM((_NBUF, _BR, _N), jnp.float32),
            pltpu.VMEM((_N, _D), jnp.bfloat16),
            pltpu.SemaphoreType.DMA((_NBUF,)),
        ],
    )(z, W, B.reshape(1, _D), dist_matrix)
    return out


# speedup vs baseline: 1.0717x; 1.0717x over previous
import jax
import jax.numpy as jnp
from jax.experimental import pallas as pl
from jax.experimental.pallas import tpu as pltpu

_N = 8192
_D = 64
_CUTOFF = 0.5
_MU = 1.0
_SIGMA = 0.5
_C2 = -1.4426950408889634 / (2.0 * _SIGMA * _SIGMA)

_BR = 512
_NBUF = 3
_NBLK = _N // _BR


def _interact_kernel(z_ref, w_ref, b_ref, dist_hbm, out_ref, buf, msg, sems):
    def copy_in(blk, slot):
        return pltpu.make_async_copy(
            dist_hbm.at[pl.ds(blk * _BR, _BR), :],
            buf.at[slot],
            sems.at[slot],
        )

    for s in range(_NBUF):
        copy_in(s, s).start()

    msg[...] = (
        jnp.dot(z_ref[...], w_ref[...], preferred_element_type=jnp.float32)
        + b_ref[...]
    ).astype(jnp.bfloat16)

    def body(blk, carry):
        slot = jax.lax.rem(blk, _NBUF)
        copy_in(blk, slot).wait()
        r = buf[slot]
        u = 1.0 / r
        t = u - 1.0 / _MU
        w = jnp.where(r < _CUTOFF, jnp.exp2(_C2 * (t * t)), 0.0).astype(jnp.bfloat16)
        out_ref[pl.ds(blk * _BR, _BR), :] = jnp.dot(
            w, msg[...], preferred_element_type=jnp.float32
        )

        @pl.when(blk + _NBUF < _NBLK)
        def _prefetch():
            copy_in(blk + _NBUF, slot).start()

        return carry

    jax.lax.fori_loop(0, _NBLK, body, 0)


def kernel(z, dist_matrix, W, B):
    out = pl.pallas_call(
        _interact_kernel,
        in_specs=[
            pl.BlockSpec((_N, _D), lambda: (0, 0)),
            pl.BlockSpec((_D, _D), lambda: (0, 0)),
            pl.BlockSpec((1, _D), lambda: (0, 0)),
            pl.BlockSpec(memory_space=pltpu.HBM),
        ],
        out_specs=pl.BlockSpec((_N, _D), lambda: (0, 0)),
        out_shape=jax.ShapeDtypeStruct((_N, _D), jnp.float32),
        scratch_shapes=[
            pltpu.VMEM((_NBUF, _BR, _N), jnp.float32),
            pltpu.VMEM((_N, _D), jnp.bfloat16),
            pltpu.SemaphoreType.DMA((_NBUF,)),
        ],
    )(z, W, B.reshape(1, _D), dist_matrix)
    return out


# speedup vs baseline: 1.1280x; 1.0525x over previous
import jax
import jax.numpy as jnp
from jax.experimental import pallas as pl
from jax.experimental.pallas import tpu as pltpu

_N = 8192
_D = 64
_CUTOFF = 0.5
_MU = 1.0
_SIGMA = 0.5
_C2 = -1.4426950408889634 / (2.0 * _SIGMA * _SIGMA)

_BR = 256
_NBUF = 3
_NBLK = _N // _BR


def _interact_kernel(z_ref, w_ref, b_ref, dist_hbm, out_ref, buf, msg, sems):
    def copy_in(blk, slot):
        return pltpu.make_async_copy(
            dist_hbm.at[pl.ds(blk * _BR, _BR), :],
            buf.at[slot],
            sems.at[slot],
        )

    for s in range(_NBUF):
        copy_in(s, s).start()

    msg[...] = (
        jnp.dot(z_ref[...], w_ref[...], preferred_element_type=jnp.float32)
        + b_ref[...]
    ).astype(jnp.bfloat16)

    def body(blk, carry):
        slot = jax.lax.rem(blk, _NBUF)
        copy_in(blk, slot).wait()
        r = buf[slot]
        u = 1.0 / r
        t = u - 1.0 / _MU
        w = jnp.where(r < _CUTOFF, jnp.exp2(_C2 * (t * t)), 0.0).astype(jnp.bfloat16)
        out_ref[pl.ds(blk * _BR, _BR), :] = jnp.dot(
            w, msg[...], preferred_element_type=jnp.float32
        )

        @pl.when(blk + _NBUF < _NBLK)
        def _prefetch():
            copy_in(blk + _NBUF, slot).start()

        return carry

    jax.lax.fori_loop(0, _NBLK, body, 0)


def kernel(z, dist_matrix, W, B):
    out = pl.pallas_call(
        _interact_kernel,
        in_specs=[
            pl.BlockSpec((_N, _D), lambda: (0, 0)),
            pl.BlockSpec((_D, _D), lambda: (0, 0)),
            pl.BlockSpec((1, _D), lambda: (0, 0)),
            pl.BlockSpec(memory_space=pltpu.HBM),
        ],
        out_specs=pl.BlockSpec((_N, _D), lambda: (0, 0)),
        out_shape=jax.ShapeDtypeStruct((_N, _D), jnp.float32),
        scratch_shapes=[
            pltpu.VMEM((_NBUF, _BR, _N), jnp.float32),
            pltpu.VMEM((_N, _D), jnp.bfloat16),
            pltpu.SemaphoreType.DMA((_NBUF,)),
        ],
    )(z, W, B.reshape(1, _D), dist_matrix)
    return out
